# baseline (speedup 1.0000x reference)
import jax
import jax.numpy as jnp
from jax import lax
from jax.experimental import pallas as pl
from jax.experimental.pallas import tpu as pltpu
from jax.experimental.pallas import tpu_sc as plsc

ROWS = 64
N = 8192
TOP_K = 512
L = 16
CH = N // L
H = 4
HH = CH // H
HSH = 7
NB = 256
HB = NB * L
NW = 32
R = 2


def _to_key(bits):
    m = lax.shift_right_arithmetic(bits, 31)
    return bits ^ (jnp.bitwise_not(m) & jnp.int32(0x7FFFFFFF))


def _sort_body(in_hbm, sel_hbm, not_hbm,
               v0, v1, ka0, ka1, kb0, kb1, ib0, ib1,
               hc0, hc1, hn0, hn1, dma_sem):
    wid = lax.axis_index("s") * 2 + lax.axis_index("c")
    lanes = lax.iota(jnp.int32, L)
    g_base = lanes * CH
    ones = jnp.ones((L,), jnp.int32)
    zeros = jnp.zeros((L,), jnp.int32)
    row0 = wid * R

    vals = (v0, v1)
    ka = (ka0, ka1)
    kb = (kb0, kb1)
    ib = (ib0, ib1)
    hc = (hc0, hc1)
    hn = (hn0, hn1)

    for r in range(R):
        pltpu.sync_copy(in_hbm.at[row0 + r], vals[r])

    def zero(hists):
        def step(i, _):
            for h in hists:
                for x in range(H):
                    h[pl.ds(i * L + x * HB, L)] = zeros
            return 0
        lax.fori_loop(0, NB, step, 0)

    def scan(hists, zhists=None):
        def step(i, carry):
            slf = [pl.ds(i * L + x * HB, L) for x in range(H)]
            slb = [pl.ds((NB - 1 - i) * L + x * HB, L) for x in range(H)]
            nxt = []
            for r in range(R):
                hf = [hists[r][s_] for s_ in slf]
                hb = [hists[r][s_] for s_ in slb]
                sf = hf[0]
                sb = hb[0]
                for x in range(1, H):
                    sf = sf + hf[x]
                    sb = sb + hb[x]
                incf = plsc.cumsum(sf)
                incb = plsc.cumsum(sb)
                cb = carry[2 * r + 1] - incb[L - 1]
                bf = incf - sf + carry[2 * r]
                bb = incb - sb + cb
                for x in range(H):
                    hists[r][slf[x]] = bf
                    hists[r][slb[x]] = bb
                    if x + 1 < H:
                        bf = bf + hf[x]
                        bb = bb + hb[x]
                nxt += [carry[2 * r] + incf[L - 1], cb]
            if zhists is not None:
                for z in zhists:
                    for s_ in slf + slb:
                        z[s_] = zeros
            return tuple(nxt)
        lax.fori_loop(0, NB // 2, step,
                      (jnp.int32(0), jnp.int32(N)) * R)

    def addr0(k):
        return (lax.shift_left(k, 4) | lanes) & jnp.int32(0xFFF)

    def addrp(k, shift, low):
        return (lax.shift_right_logical(k, shift - 4) & jnp.int32(0xFF0)) | low

    def halfbit(off):
        return lax.shift_left(
            lax.shift_right_logical(off, HSH) & jnp.int32(H - 1), 12)

    def dest(off):
        own = lax.shift_right_logical(off, 9)
        q = lax.shift_left(off & jnp.int32(CH - 1), 4) | own
        return own, q

    RH = tuple((r, x) for r in range(R) for x in range(H))

    def s0_step(t, _):
        g = [g_base + t + x * HH for (r, x) in RH]
        v = [plsc.load_gather(vals[r], [g[j]]) for j, (r, x) in enumerate(RH)]
        k = [_to_key(lax.bitcast_convert_type(vj + jnp.float32(0.0),
                                              jnp.int32)) for vj in v]
        a = [addr0(k[j]) + x * HB for j, (r, x) in enumerate(RH)]
        for j, (r, x) in enumerate(RH):
            ka[r][pl.ds((t + x * HH) * L, L)] = k[j]
        for j, (r, x) in enumerate(RH):
            plsc.addupdate_scatter(hc[r], [a[j]], ones)
        return 0

    def scat0():
        def step(t, _):
            sl = [pl.ds((t + x * HH) * L, L) for (r, x) in RH]
            g = [g_base + t + x * HH for (r, x) in RH]
            k = [ka[r][sl[j]] for j, (r, x) in enumerate(RH)]
            a = [addr0(k[j]) + x * HB for j, (r, x) in enumerate(RH)]
            off = [plsc.load_gather(hc[r], [a[j]])
                   for j, (r, x) in enumerate(RH)]
            oq = [dest(o) for o in off]
            a2 = [addrp(k[j], 8, oq[j][0]) | halfbit(off[j])
                  for j in range(len(RH))]
            for j, (r, x) in enumerate(RH):
                plsc.store_scatter(kb[r], [oq[j][1]], k[j])
            for j, (r, x) in enumerate(RH):
                plsc.store_scatter(ib[r], [oq[j][1]], g[j])
            for j, (r, x) in enumerate(RH):
                plsc.addupdate_scatter(hc[r], [a[j]], ones)
            for j, (r, x) in enumerate(RH):
                plsc.addupdate_scatter(hn[r], [a2[j]], ones)
            return 0
        lax.fori_loop(0, HH, step, 0)

    def scat1():
        def step(t, _):
            sl = [pl.ds((t + x * HH) * L, L) for (r, x) in RH]
            k = [kb[r][sl[j]] for j, (r, x) in enumerate(RH)]
            i = [ib[r][sl[j]] for j, (r, x) in enumerate(RH)]
            a = [addrp(k[j], 8, lanes) + x * HB
                 for j, (r, x) in enumerate(RH)]
            off = [plsc.load_gather(hn[r], [a[j]])
                   for j, (r, x) in enumerate(RH)]
            oq = [dest(o) for o in off]
            a2 = [addrp(k[j], 16, oq[j][0]) | halfbit(off[j])
                  for j in range(len(RH))]
            pk = [lax.shift_left(lax.shift_right_logical(k[j], 16), 13)
                  | i[j] for j in range(len(RH))]
            for j, (r, x) in enumerate(RH):
                plsc.store_scatter(ka[r], [oq[j][1]], pk[j])
            for j, (r, x) in enumerate(RH):
                plsc.addupdate_scatter(hn[r], [a[j]], ones)
            for j, (r, x) in enumerate(RH):
                plsc.addupdate_scatter(hc[r], [a2[j]], ones)
            return 0
        lax.fori_loop(0, HH, step, 0)

    def addr_pk(p_, sh, low):
        return (lax.shift_right_logical(p_, sh) & jnp.int32(0xFF0)) | low

    def scat2():
        def step(t, _):
            sl = [pl.ds((t + x * HH) * L, L) for (r, x) in RH]
            p_ = [ka[r][sl[j]] for j, (r, x) in enumerate(RH)]
            a = [addr_pk(p_[j], 9, lanes) + x * HB
                 for j, (r, x) in enumerate(RH)]
            off = [plsc.load_gather(hc[r], [a[j]])
                   for j, (r, x) in enumerate(RH)]
            oq = [dest(o) for o in off]
            a2 = [addr_pk(p_[j], 17, oq[j][0]) | halfbit(off[j])
                  for j in range(len(RH))]
            for j, (r, x) in enumerate(RH):
                plsc.store_scatter(kb[r], [oq[j][1]], p_[j])
            for j, (r, x) in enumerate(RH):
                plsc.addupdate_scatter(hc[r], [a[j]], ones)
            for j, (r, x) in enumerate(RH):
                plsc.addupdate_scatter(hn[r], [a2[j]], ones)
            return 0
        lax.fori_loop(0, HH, step, 0)

    def scat3():
        def step(t, _):
            sl = [pl.ds((t + x * HH) * L, L) for (r, x) in RH]
            p_ = [kb[r][sl[j]] for j, (r, x) in enumerate(RH)]
            a = [addr_pk(p_[j], 17, lanes) + x * HB
                 for j, (r, x) in enumerate(RH)]
            off = [plsc.load_gather(hn[r], [a[j]])
                   for j, (r, x) in enumerate(RH)]
            for j, (r, x) in enumerate(RH):
                plsc.store_scatter(ib[r], [off[j]],
                                   p_[j] & jnp.int32(0x1FFF))
            for j, (r, x) in enumerate(RH):
                plsc.addupdate_scatter(hn[r], [a[j]], ones)
            return 0
        lax.fori_loop(0, HH, step, 0)

    zero(hc)
    lax.fori_loop(0, HH, s0_step, 0)
    scan(hc, zhists=hn)
    scat0()
    scan(hn, zhists=hc)
    scat1()
    scan(hc, zhists=hn)
    scat2()
    scan(hn)
    scat3()

    for r in range(R):
        pltpu.sync_copy(ib[r].at[pl.ds(0, TOP_K)], sel_hbm.at[row0 + r])
        pltpu.sync_copy(ib[r].at[pl.ds(TOP_K, N - TOP_K)],
                        not_hbm.at[row0 + r])


@jax.jit
def _run(inputs):
    mesh = plsc.VectorSubcoreMesh(core_axis_name="c", subcore_axis_name="s")
    f = pl.kernel(
        _sort_body,
        out_type=(
            jax.ShapeDtypeStruct((ROWS, TOP_K), jnp.int32),
            jax.ShapeDtypeStruct((ROWS, N - TOP_K), jnp.int32),
        ),
        mesh=mesh,
        scratch_types=[
            pltpu.VMEM((N,), jnp.float32),
            pltpu.VMEM((N,), jnp.float32),
            pltpu.VMEM((N,), jnp.int32),
            pltpu.VMEM((N,), jnp.int32),
            pltpu.VMEM((N,), jnp.int32),
            pltpu.VMEM((N,), jnp.int32),
            pltpu.VMEM((N,), jnp.int32),
            pltpu.VMEM((N,), jnp.int32),
            pltpu.VMEM((H * NB * L,), jnp.int32),
            pltpu.VMEM((H * NB * L,), jnp.int32),
            pltpu.VMEM((H * NB * L,), jnp.int32),
            pltpu.VMEM((H * NB * L,), jnp.int32),
            pltpu.SemaphoreType.DMA,
        ],
        compiler_params=pltpu.CompilerParams(needs_layout_passes=False),
    )
    return f(inputs)


def kernel(inputs):
    return _run(inputs)
